# Initial kernel scaffold; baseline (speedup 1.0000x reference)
import jax
import jax.numpy as jnp
from jax.experimental import pallas as pl


def kernel(inputs, embedding):
    raise NotImplementedError("write your pallas kernel here")


# speedup vs baseline: 1.0810x; 1.0810x over previous
import functools

import jax
import jax.numpy as jnp
from jax.experimental import pallas as pl
from jax.experimental.pallas import tpu as pltpu

_K = 1024
_C = 64
_N = 16 * 48 * 48
_MBLK = 1152
_GRID = _N // _MBLK
_TOTAL = float(_N * _C)


def _vq_body(x_ref, in2_ref, embt_ref, emb_ref,
             st_ref, loss_ref, perp_ref, sse_acc, cnt_acc):
    g = pl.program_id(0)
    x = x_ref[...]
    emb = emb_ref[...]
    embt = embt_ref[...]

    x2 = jnp.sum(x * x, axis=1, keepdims=True)
    m = jax.lax.dot_general(x, embt, (((1,), (0,)), ((), ())),
                            preferred_element_type=jnp.float32)
    e2 = jnp.sum(emb * emb, axis=1)
    d = (x2 - 2.0 * m) + e2[None, :]
    iota = jax.lax.broadcasted_iota(jnp.int32, (_MBLK, _K), 1)
    dmin = jnp.min(d, axis=1, keepdims=True)
    idx = jnp.min(jnp.where(d == dmin, iota, _K), axis=1)

    enc = (iota == idx[:, None]).astype(jnp.float32)
    q = jax.lax.dot_general(enc, emb, (((1,), (0,)), ((), ())),
                            preferred_element_type=jnp.float32)

    xin = in2_ref[...]
    diff = q - xin
    st_ref[...] = xin + diff

    sse = jnp.sum(diff * diff)
    cnt = jnp.sum(enc, axis=0).reshape(1, _K)

    prev_sse = jnp.where(g == 0, 0.0, sse_acc[0, 0])
    sse_acc[0, 0] = prev_sse + sse
    prev_cnt = jnp.where(g == 0, jnp.zeros_like(cnt), cnt_acc[...])
    cnt_acc[...] = prev_cnt + cnt

    @pl.when(g == _GRID - 1)
    def _():
        mse = sse_acc[0, 0] / _TOTAL
        loss_ref[...] = jnp.full((1, 1), mse + 0.25 * mse, jnp.float32)
        p = cnt_acc[...] / float(_N)
        ent = -jnp.sum(p * jnp.log(p + 1e-10))
        perp_ref[...] = jnp.full((1, 1), jnp.exp(ent), jnp.float32)


@jax.jit
def kernel(inputs, embedding):
    B, C, H, W = inputs.shape
    flat = jnp.transpose(inputs, (0, 2, 3, 1)).reshape(-1, C)
    in2d = inputs.reshape(-1, C)
    embt = embedding.T

    st2d, loss, perp = pl.pallas_call(
        _vq_body,
        grid=(_GRID,),
        in_specs=[
            pl.BlockSpec((_MBLK, _C), lambda g: (g, 0)),
            pl.BlockSpec((_MBLK, _C), lambda g: (g, 0)),
            pl.BlockSpec((_C, _K), lambda g: (0, 0)),
            pl.BlockSpec((_K, _C), lambda g: (0, 0)),
        ],
        out_specs=[
            pl.BlockSpec((_MBLK, _C), lambda g: (g, 0)),
            pl.BlockSpec((1, 1), lambda g: (0, 0)),
            pl.BlockSpec((1, 1), lambda g: (0, 0)),
        ],
        out_shape=[
            jax.ShapeDtypeStruct((_N, _C), jnp.float32),
            jax.ShapeDtypeStruct((1, 1), jnp.float32),
            jax.ShapeDtypeStruct((1, 1), jnp.float32),
        ],
        scratch_shapes=[
            pltpu.SMEM((1, 1), jnp.float32),
            pltpu.VMEM((1, _K), jnp.float32),
        ],
    )(flat, in2d, embt, embedding)

    return st2d.reshape(inputs.shape), loss[0, 0], perp[0, 0]
